# baseline (speedup 1.0000x reference)
import functools

import jax
import jax.numpy as jnp
from jax import lax
from jax.experimental import pallas as pl
from jax.experimental.pallas import tpu as pltpu
from jax.experimental.pallas import tpu_sc as plsc

NUM_NODES = 50000
NUM_EDGES = 1600000
BATCH = 32

NC = 2
NS = 16
L = 16

N_QUARTERS = 4
N_GROUPS = 8
EDGE_Q = NUM_EDGES // N_QUARTERS
CHUNK = 3200
N_CHUNKS = EDGE_Q // CHUNK

_ILV = plsc.PackFormat.INTERLEAVED


def _sc_body(xp_hbm, j_hbm, ep_hbm, out_hbm,
             xpa, xpb, epc, jc, accv, sem0, sem1):
    cid = lax.axis_index("c")
    sid = lax.axis_index("s")
    wid = sid * NC + cid
    g = lax.rem(wid, N_GROUPS)
    q = wid // N_GROUPS
    edge_base = q * EDGE_Q

    pltpu.sync_copy(xp_hbm.at[2 * g], xpa)
    pltpu.sync_copy(xp_hbm.at[2 * g + 1], xpb)

    sems = (sem0, sem1)

    def issue(ci, slot):
        base = edge_base + ci * CHUNK
        pltpu.async_copy(ep_hbm.at[pl.ds(base, CHUNK)], epc.at[slot], sems[slot])
        pltpu.async_copy(j_hbm.at[pl.ds(base, CHUNK)], jc.at[slot], sems[slot])

    def drain(slot):
        pltpu.make_async_copy(ep_hbm.at[pl.ds(0, CHUNK)], epc.at[slot], sems[slot]).wait()
        pltpu.make_async_copy(j_hbm.at[pl.ds(0, CHUNK)], jc.at[slot], sems[slot]).wait()

    def group(slot, off, acc):
        a0, a1, a2, a3 = acc
        w = epc[slot, pl.ds(off, L)]
        jv = jc[slot, pl.ds(off, L)]
        ii = w & 0xFFFF
        jj = lax.shift_right_logical(w, 16)
        wia = plsc.load_gather(xpa, [ii])
        wja = plsc.load_gather(xpa, [jj])
        wib = plsc.load_gather(xpb, [ii])
        wjb = plsc.load_gather(xpb, [jj])
        jvp = plsc.pack(jv, jv, format=_ILV)
        pa = plsc.bitcast(wia, jnp.bfloat16) * plsc.bitcast(wja, jnp.bfloat16)
        pb = plsc.bitcast(wib, jnp.bfloat16) * plsc.bitcast(wjb, jnp.bfloat16)
        p0, p1 = plsc.unpack(pa * jvp, format=_ILV)
        p2, p3 = plsc.unpack(pb * jvp, format=_ILV)
        return (a0 + p0, a1 + p1, a2 + p2, a3 + p3)

    def compute(slot, acc):
        accA, accB = acc

        @plsc.parallel_loop(0, CHUNK, 2 * L, unroll=4, carry=(accA, accB))
        def loop(off, carry):
            cA, cB = carry
            return (group(slot, off, cA), group(slot, off + L, cB))

        return loop

    zero = jnp.zeros((L,), jnp.float32)
    acc = ((zero, zero, zero, zero), (zero, zero, zero, zero))

    issue(0, 0)

    def pair_body(p, acc):
        ci = 2 * p
        issue(ci + 1, 1)
        drain(0)
        acc = compute(0, acc)
        issue(ci + 2, 0)
        drain(1)
        acc = compute(1, acc)
        return acc

    acc = lax.fori_loop(0, (N_CHUNKS - 1) // 2, pair_body, acc)
    drain(0)
    acc = compute(0, acc)

    accA, accB = acc
    for k in range(4):
        accv[k, :] = accA[k] + accB[k]
    for k in range(4):
        pltpu.sync_copy(accv.at[k], out_hbm.at[g, k, q])


@jax.jit
def _sc_energy(xp, j, ep):
    mesh = plsc.VectorSubcoreMesh(core_axis_name="c", subcore_axis_name="s",
                                  num_cores=NC, num_subcores=NS)
    run = pl.kernel(
        _sc_body,
        out_type=jax.ShapeDtypeStruct((N_GROUPS, 4, N_QUARTERS, L),
                                      jnp.float32),
        mesh=mesh,
        compiler_params=pltpu.CompilerParams(needs_layout_passes=False),
        scratch_types=[
            pltpu.VMEM((NUM_NODES,), jnp.float32),
            pltpu.VMEM((NUM_NODES,), jnp.float32),
            pltpu.VMEM((2, CHUNK), jnp.int32),
            pltpu.VMEM((2, CHUNK), jnp.float32),
            pltpu.VMEM((4, L), jnp.float32),
            pltpu.SemaphoreType.DMA,
            pltpu.SemaphoreType.DMA,
        ],
    )
    return run(xp, j, ep)


def _xh_body(x_ref, h_ref, out_ref):
    out_ref[...] = jnp.sum(x_ref[...] * h_ref[...], axis=1, keepdims=True)


@jax.jit
def _xh_matvec(x, h):
    return pl.pallas_call(
        _xh_body,
        out_shape=jax.ShapeDtypeStruct((BATCH, 1), jnp.float32),
    )(x, h.reshape(1, NUM_NODES))


def kernel(x, h, J, edge_idx_i, edge_idx_j):
    xr = x.astype(jnp.bfloat16).reshape(BATCH // 2, 2, NUM_NODES)
    xr = jnp.swapaxes(xr, 1, 2)
    xp = lax.bitcast_convert_type(xr, jnp.float32)

    ei = edge_idx_i.astype(jnp.int32)
    ej = edge_idx_j.astype(jnp.int32)
    ep = ei | (ej << 16)

    partials = _sc_energy(xp, J, ep)
    xh = _xh_matvec(x, h)
    return partials.reshape(BATCH, N_QUARTERS * L).sum(axis=-1) + xh[:, 0]


# speedup vs baseline: 1.0088x; 1.0088x over previous
import functools

import jax
import jax.numpy as jnp
from jax import lax
from jax.experimental import pallas as pl
from jax.experimental.pallas import tpu as pltpu
from jax.experimental.pallas import tpu_sc as plsc

NUM_NODES = 50000
NUM_EDGES = 1600000
BATCH = 32

NC = 2
NS = 16
L = 16

N_QUARTERS = 4
N_GROUPS = 8
EDGE_Q = NUM_EDGES // N_QUARTERS
CHUNK = 3200
N_CHUNKS = EDGE_Q // CHUNK

_ILV = plsc.PackFormat.INTERLEAVED


def _sc_body(xp_hbm, j_hbm, ep_hbm, out_hbm,
             xpa, xpb, epc, jc, accv, sem0, sem1):
    cid = lax.axis_index("c")
    sid = lax.axis_index("s")
    wid = sid * NC + cid
    g = lax.rem(wid, N_GROUPS)
    q = wid // N_GROUPS
    edge_base = q * EDGE_Q

    pltpu.sync_copy(xp_hbm.at[2 * g], xpa)
    pltpu.sync_copy(xp_hbm.at[2 * g + 1], xpb)

    sems = (sem0, sem1)

    def issue(ci, slot):
        base = edge_base + ci * CHUNK
        pltpu.async_copy(ep_hbm.at[pl.ds(base, CHUNK)], epc.at[slot], sems[slot])
        pltpu.async_copy(j_hbm.at[pl.ds(base, CHUNK)], jc.at[slot], sems[slot])

    def drain(slot):
        pltpu.make_async_copy(ep_hbm.at[pl.ds(0, CHUNK)], epc.at[slot], sems[slot]).wait()
        pltpu.make_async_copy(j_hbm.at[pl.ds(0, CHUNK)], jc.at[slot], sems[slot]).wait()

    def group(slot, off, acc):
        a0, a1, a2, a3 = acc
        w = epc[slot, pl.ds(off, L)]
        jv = jc[slot, pl.ds(off, L)]
        ii = w & 0xFFFF
        jj = lax.shift_right_logical(w, 16)
        wia = plsc.load_gather(xpa, [ii])
        wja = plsc.load_gather(xpa, [jj])
        wib = plsc.load_gather(xpb, [ii])
        wjb = plsc.load_gather(xpb, [jj])
        jvp = plsc.pack(jv, jv, format=_ILV)
        pa = plsc.bitcast(wia, jnp.bfloat16) * plsc.bitcast(wja, jnp.bfloat16)
        pb = plsc.bitcast(wib, jnp.bfloat16) * plsc.bitcast(wjb, jnp.bfloat16)
        p0, p1 = plsc.unpack(pa * jvp, format=_ILV)
        p2, p3 = plsc.unpack(pb * jvp, format=_ILV)
        return (a0 + p0, a1 + p1, a2 + p2, a3 + p3)

    def compute(slot, acc):
        @plsc.parallel_loop(0, CHUNK, L, unroll=8, carry=acc)
        def loop(off, acc):
            return group(slot, off, acc)

        return loop

    zero = jnp.zeros((L,), jnp.float32)
    acc = (zero, zero, zero, zero)

    issue(0, 0)

    def pair_body(p, acc):
        ci = 2 * p
        issue(ci + 1, 1)
        drain(0)
        acc = compute(0, acc)
        issue(ci + 2, 0)
        drain(1)
        acc = compute(1, acc)
        return acc

    acc = lax.fori_loop(0, (N_CHUNKS - 1) // 2, pair_body, acc)
    drain(0)
    acc = compute(0, acc)

    for k in range(4):
        accv[k, :] = acc[k]
    for k in range(4):
        pltpu.sync_copy(accv.at[k], out_hbm.at[g, k, q])


@jax.jit
def _sc_energy(xp, j, ep):
    mesh = plsc.VectorSubcoreMesh(core_axis_name="c", subcore_axis_name="s",
                                  num_cores=NC, num_subcores=NS)
    run = pl.kernel(
        _sc_body,
        out_type=jax.ShapeDtypeStruct((N_GROUPS, 4, N_QUARTERS, L),
                                      jnp.float32),
        mesh=mesh,
        compiler_params=pltpu.CompilerParams(needs_layout_passes=False),
        scratch_types=[
            pltpu.VMEM((NUM_NODES,), jnp.float32),
            pltpu.VMEM((NUM_NODES,), jnp.float32),
            pltpu.VMEM((2, CHUNK), jnp.int32),
            pltpu.VMEM((2, CHUNK), jnp.float32),
            pltpu.VMEM((4, L), jnp.float32),
            pltpu.SemaphoreType.DMA,
            pltpu.SemaphoreType.DMA,
        ],
    )
    return run(xp, j, ep)


def _xh_body(x_ref, h_ref, out_ref):
    out_ref[...] = jnp.sum(x_ref[...] * h_ref[...], axis=1, keepdims=True)


@jax.jit
def _xh_matvec(x, h):
    return pl.pallas_call(
        _xh_body,
        out_shape=jax.ShapeDtypeStruct((BATCH, 1), jnp.float32),
    )(x, h.reshape(1, NUM_NODES))


def kernel(x, h, J, edge_idx_i, edge_idx_j):
    xr = x.astype(jnp.bfloat16).reshape(BATCH // 2, 2, NUM_NODES)
    xr = jnp.swapaxes(xr, 1, 2)
    xp = lax.bitcast_convert_type(xr, jnp.float32)

    ei = edge_idx_i.astype(jnp.int32)
    ej = edge_idx_j.astype(jnp.int32)
    ep = ei | (ej << 16)

    partials = _sc_energy(xp, J, ep)
    xh = _xh_matvec(x, h)
    return partials.reshape(BATCH, N_QUARTERS * L).sum(axis=-1) + xh[:, 0]
